# Initial kernel scaffold; baseline (speedup 1.0000x reference)
import jax
import jax.numpy as jnp
from jax.experimental import pallas as pl


def kernel(x, edge_index, edge_attr, Wq, bq, Wk, bk, Wv, bv, Ws, bs):
    raise NotImplementedError("write your pallas kernel here")


# speedup vs baseline: 10.6238x; 10.6238x over previous
import functools

import jax
import jax.numpy as jnp
from jax import lax
from jax.experimental import pallas as pl
from jax.experimental.pallas import tpu as pltpu
from jax.experimental.pallas import tpu_sc as plsc

N = 10000
E = 160000
D = 256
H = 8
C = 32
HC = H * C

NC = 2
NS = 16
NW = NC * NS

CH = 128
NCHUNK = E // CH
T_ITERS = -(-NCHUNK // NW)
T2_ITERS = -(-NCHUNK // NS)

STRIPE = 624
SPAN = 640


def _proj_body(x_ref, w_ref, b_ref, q_ref, k_ref, vl_ref, vr_ref):
    y = jnp.dot(x_ref[...], w_ref[...], preferred_element_type=jnp.float32)
    y = y + b_ref[...]
    q_ref[...] = y[:, 0:256]
    k_ref[...] = y[:, 256:512]
    vl_ref[...] = y[:, 512:640]
    vr_ref[...] = y[:, 640:768]


def _proj(x, w, b):
    bn = 1000
    grid = (N // bn,)
    return pl.pallas_call(
        _proj_body,
        grid=grid,
        in_specs=[
            pl.BlockSpec((bn, D), lambda i: (i, 0)),
            pl.BlockSpec((D, 3 * HC), lambda i: (0, 0)),
            pl.BlockSpec((1, 3 * HC), lambda i: (0, 0)),
        ],
        out_specs=[
            pl.BlockSpec((bn, HC), lambda i: (i, 0)),
            pl.BlockSpec((bn, HC), lambda i: (i, 0)),
            pl.BlockSpec((bn, 128), lambda i: (i, 0)),
            pl.BlockSpec((bn, 128), lambda i: (i, 0)),
        ],
        out_shape=[
            jax.ShapeDtypeStruct((N, HC), jnp.float32),
            jax.ShapeDtypeStruct((N, HC), jnp.float32),
            jax.ShapeDtypeStruct((N, 128), jnp.float32),
            jax.ShapeDtypeStruct((N, 128), jnp.float32),
        ],
    )(x, w, b)


def _gather_body(q_hbm, k_hbm, dst_hbm, src_hbm, qi_hbm, kj_hbm,
                 idxd, idxs, qb, kb, sem):
    wid = lax.axis_index("s") * NC + lax.axis_index("c")

    def chunk(t, carry):
        cid = wid + NW * t

        @pl.when(cid < NCHUNK)
        def _():
            base = cid * CH
            pltpu.sync_copy(dst_hbm.at[pl.ds(base, CH)], idxd)
            pltpu.sync_copy(src_hbm.at[pl.ds(base, CH)], idxs)
            pltpu.async_copy(q_hbm.at[idxd], qb, sem).wait()
            pltpu.async_copy(k_hbm.at[idxs], kb, sem).wait()
            pltpu.sync_copy(qb, qi_hbm.at[pl.ds(base, CH)])
            pltpu.sync_copy(kb, kj_hbm.at[pl.ds(base, CH)])

        return carry

    lax.fori_loop(0, T_ITERS, chunk, 0)


def _gather_qk(q, k, dst, src):
    mesh = plsc.VectorSubcoreMesh(
        core_axis_name="c", subcore_axis_name="s",
        num_cores=NC, num_subcores=NS)
    fn = pl.kernel(
        _gather_body,
        out_type=[
            jax.ShapeDtypeStruct((E, HC), jnp.float32),
            jax.ShapeDtypeStruct((E, HC), jnp.float32),
        ],
        mesh=mesh,
        scratch_types=[
            pltpu.VMEM((CH,), jnp.int32),
            pltpu.VMEM((CH,), jnp.int32),
            pltpu.VMEM((CH, HC), jnp.float32),
            pltpu.VMEM((CH, HC), jnp.float32),
            pltpu.SemaphoreType.DMA,
        ],
    )
    return fn(q, k, dst, src)


def _alpha_body(qi_ref, kj_ref, g_ref, m_ref, rb_ref, sl_ref, sr_ref):
    p = qi_ref[...] * kj_ref[...]
    a = jnp.dot(p, g_ref[...], preferred_element_type=jnp.float32)
    s = jnp.exp(a * (1.0 / (C ** 0.5))) * m_ref[...]
    sb = jnp.dot(s, rb_ref[...], preferred_element_type=jnp.float32)
    sl_ref[...] = sb[:, :128]
    sr_ref[...] = sb[:, 128:]


def _alpha(qi, kj, g, m, rb):
    be = 2000
    grid = (E // be,)
    return pl.pallas_call(
        _alpha_body,
        grid=grid,
        in_specs=[
            pl.BlockSpec((be, HC), lambda i: (i, 0)),
            pl.BlockSpec((be, HC), lambda i: (i, 0)),
            pl.BlockSpec((HC, 16), lambda i: (0, 0)),
            pl.BlockSpec((1, 16), lambda i: (0, 0)),
            pl.BlockSpec((16, HC), lambda i: (0, 0)),
        ],
        out_specs=[
            pl.BlockSpec((be, 128), lambda i: (i, 0)),
            pl.BlockSpec((be, 128), lambda i: (i, 0)),
        ],
        out_shape=[
            jax.ShapeDtypeStruct((E, 128), jnp.float32),
            jax.ShapeDtypeStruct((E, 128), jnp.float32),
        ],
    )(qi, kj, g, m, rb)


def _scatter_body(vl_hbm, vr_hbm, eal_hbm, ear_hbm, sl_hbm, sr_hbm,
                  dst_hbm, src_hbm,
                  accl_hbm, accr_hbm, denl_hbm, denr_hbm,
                  idxs, idxd, vbuf, eabuf, slbuf,
                  acc_sh, sem):
    core = lax.axis_index("c")
    sid = lax.axis_index("s")
    wid = sid * NC + core
    row0 = sid * STRIPE

    def zero_acc():
        def zrow(r, carry):
            for j in range(8):
                vbuf[r, pl.ds(16 * j, 16)] = jnp.zeros((16,), jnp.float32)
            return carry

        lax.fori_loop(0, 64, zrow, 0)
        for j in range(SPAN // 64):
            pltpu.sync_copy(vbuf.at[pl.ds(0, 64)],
                            acc_sh.at[pl.ds(row0 + 64 * j, 64)])

    def edge_pass(half):
        v_hbm = (vl_hbm, vr_hbm)[half]
        ea_hbm = (eal_hbm, ear_hbm)[half]
        sh_hbm = (sl_hbm, sr_hbm)[half]

        def chunk(t, carry):
            cid = sid + NS * t

            @pl.when(cid < NCHUNK)
            def _():
                base = cid * CH
                pltpu.sync_copy(dst_hbm.at[pl.ds(base, CH)], idxd)
                pltpu.sync_copy(src_hbm.at[pl.ds(base, CH)], idxs)
                pltpu.async_copy(v_hbm.at[idxs], vbuf, sem).wait()

                for hh in range(8):
                    pltpu.sync_copy(
                        ea_hbm.at[pl.ds(base + 16 * hh, 16)], eabuf)
                    pltpu.sync_copy(
                        sh_hbm.at[pl.ds(base + 16 * hh, 16)], slbuf)

                    def edge(i, carry2):
                        ie = i + 16 * hh
                        for g in range(8):
                            col = 16 * g
                            vbuf[ie, pl.ds(col, 16)] = (
                                vbuf[ie, pl.ds(col, 16)]
                                + eabuf[i, pl.ds(col, 16)]
                            ) * slbuf[i, pl.ds(col, 16)]
                        return carry2

                    lax.fori_loop(0, 16, edge, 0)

                pltpu.sync_copy(vbuf, acc_sh.at[idxd], add=True)

            return carry

        lax.fori_loop(0, T2_ITERS, chunk, 0)

    def den_pass(half):
        sh_hbm = (sl_hbm, sr_hbm)[half]

        def chunk(t, carry):
            cid = sid + NS * t

            @pl.when(cid < NCHUNK)
            def _():
                base = cid * CH
                pltpu.sync_copy(dst_hbm.at[pl.ds(base, CH)], idxd)
                pltpu.sync_copy(sh_hbm.at[pl.ds(base, CH)], vbuf)
                pltpu.sync_copy(vbuf, acc_sh.at[idxd], add=True)

            return carry

        lax.fori_loop(0, T2_ITERS, chunk, 0)

    zero_acc()
    plsc.subcore_barrier()

    @pl.when(core == 0)
    def _():
        edge_pass(0)

    @pl.when(core == 1)
    def _():
        edge_pass(1)

    plsc.subcore_barrier()

    @pl.when(core == 0)
    def _():
        pltpu.sync_copy(acc_sh.at[pl.ds(row0, SPAN)],
                        accl_hbm.at[pl.ds(row0, SPAN)])

    @pl.when(core == 1)
    def _():
        pltpu.sync_copy(acc_sh.at[pl.ds(row0, SPAN)],
                        accr_hbm.at[pl.ds(row0, SPAN)])

    plsc.subcore_barrier()

    zero_acc()
    plsc.subcore_barrier()

    @pl.when(core == 0)
    def _():
        den_pass(0)

    @pl.when(core == 1)
    def _():
        den_pass(1)

    plsc.subcore_barrier()

    @pl.when(core == 0)
    def _():
        pltpu.sync_copy(acc_sh.at[pl.ds(row0, SPAN)],
                        denl_hbm.at[pl.ds(row0, SPAN)])

    @pl.when(core == 1)
    def _():
        pltpu.sync_copy(acc_sh.at[pl.ds(row0, SPAN)],
                        denr_hbm.at[pl.ds(row0, SPAN)])


def _scatter(vl, vr, eal, ear, sl, sr, dst, src):
    mesh = plsc.VectorSubcoreMesh(
        core_axis_name="c", subcore_axis_name="s",
        num_cores=NC, num_subcores=NS)
    fn = pl.kernel(
        _scatter_body,
        out_type=[
            jax.ShapeDtypeStruct((N, 128), jnp.float32),
            jax.ShapeDtypeStruct((N, 128), jnp.float32),
            jax.ShapeDtypeStruct((N, 128), jnp.float32),
            jax.ShapeDtypeStruct((N, 128), jnp.float32),
        ],
        mesh=mesh,
        scratch_types=[
            pltpu.VMEM((CH,), jnp.int32),
            pltpu.VMEM((CH,), jnp.int32),
            pltpu.VMEM((CH, 128), jnp.float32),
            pltpu.VMEM((16, 128), jnp.float32),
            pltpu.VMEM((16, 128), jnp.float32),
            pltpu.VMEM_SHARED((N, 128), jnp.float32),
            pltpu.SemaphoreType.DMA,
        ],
    )
    return fn(vl, vr, eal, ear, sl, sr, dst, src)


def _out_body(al_ref, ar_ref, dl_ref, dr_ref, x_ref, ws_ref, bs_ref, o_ref):
    skip = jnp.dot(x_ref[...], ws_ref[...],
                   preferred_element_type=jnp.float32) + bs_ref[...]
    acc = jnp.concatenate([al_ref[...], ar_ref[...]], axis=1)
    dexp = jnp.concatenate([dl_ref[...], dr_ref[...]], axis=1)
    o_ref[...] = acc / (dexp + 1e-16) + skip


def _finalize(accl, accr, denl, denr, x, ws, bs):
    bn = 1000
    grid = (N // bn,)
    return pl.pallas_call(
        _out_body,
        grid=grid,
        in_specs=[
            pl.BlockSpec((bn, 128), lambda i: (i, 0)),
            pl.BlockSpec((bn, 128), lambda i: (i, 0)),
            pl.BlockSpec((bn, 128), lambda i: (i, 0)),
            pl.BlockSpec((bn, 128), lambda i: (i, 0)),
            pl.BlockSpec((bn, D), lambda i: (i, 0)),
            pl.BlockSpec((D, HC), lambda i: (0, 0)),
            pl.BlockSpec((1, HC), lambda i: (0, 0)),
        ],
        out_specs=pl.BlockSpec((bn, HC), lambda i: (i, 0)),
        out_shape=jax.ShapeDtypeStruct((N, HC), jnp.float32),
    )(accl, accr, denl, denr, x, ws, bs)


@jax.jit
def kernel(x, edge_index, edge_attr, Wq, bq, Wk, bk, Wv, bv, Ws, bs):
    src = edge_index[0]
    dst = edge_index[1]
    ea_full = edge_attr.reshape(E, HC)
    eal = ea_full[:, :128]
    ear = ea_full[:, 128:]

    w = jnp.concatenate([Wq, Wk, Wv], axis=1)
    b = jnp.concatenate([bq, bk, bv]).reshape(1, 3 * HC)

    q, k, vl, vr = _proj(x, w, b)

    qi, kj = _gather_qk(q, k, dst, src)

    g = jnp.kron(jnp.eye(H, dtype=jnp.float32),
                 jnp.ones((C, 1), dtype=jnp.float32))
    g = jnp.concatenate([g, jnp.zeros((HC, 8), jnp.float32)], axis=1)
    mask = jnp.concatenate(
        [jnp.ones((1, H), jnp.float32), jnp.zeros((1, 8), jnp.float32)],
        axis=1)
    bexp = jnp.kron(jnp.eye(H, dtype=jnp.float32),
                    jnp.ones((1, C), dtype=jnp.float32))
    bexp = jnp.concatenate([bexp, jnp.zeros((8, HC), jnp.float32)], axis=0)

    sl, sr = _alpha(qi, kj, g, mask, bexp)

    accl, accr, denl, denr = _scatter(vl, vr, eal, ear, sl, sr, dst, src)

    return _finalize(accl, accr, denl, denr, x, Ws, bs.reshape(1, HC))
